# baseline (speedup 1.0000x reference)
import functools

import jax
import jax.numpy as jnp
from jax import lax
from jax.experimental import pallas as pl
from jax.experimental.pallas import tpu as pltpu

_BM0 = 64
_BM = 256
_NSLOT = 2
_KR = 2560


def _fused_kernel(l_hbm, x0b_ref, w_ref, b_ref, out_ref, lb_hbm,
                  x1b_ref, x2b_ref, lbr_ref, lf_buf, sb_buf, lb_buf,
                  lf_sem, st_sem, lb_sem, *, n, kr, bm0, bm, f_out, pool):
    nm0 = n // bm0
    nr0 = kr // bm0
    nm = n // bm
    nr = kr // bm
    ns = nm - nr

    def load_f32(i, slot):
        return pltpu.make_async_copy(
            l_hbm.at[pl.ds(i * bm0, bm0), :], lf_buf.at[slot],
            lf_sem.at[slot])

    def store_b(i, slot):
        return pltpu.make_async_copy(
            sb_buf.at[slot], lb_hbm.at[pl.ds(i * bm0 - kr, bm0), :],
            st_sem.at[slot])

    def load_b(j, slot):
        return pltpu.make_async_copy(
            lb_hbm.at[pl.ds(j * bm, bm), :], lb_buf.at[slot],
            lb_sem.at[slot])

    load_f32(0, 0).start()
    load_f32(1, 1).start()

    def p0_step(i, lband):
        y = jnp.dot(lband, x0b_ref[...], preferred_element_type=jnp.float32)
        x1b_ref[pl.ds(i * bm0, bm0), :] = y.astype(jnp.bfloat16)

    def phase0_res(i, carry):
        slot = lax.rem(i, 2)
        load_f32(i, slot).wait()
        lbr_ref[pl.ds(i * bm0, bm0), :] = lf_buf[slot].astype(jnp.bfloat16)
        p0_step(i, lbr_ref[pl.ds(i * bm0, bm0), :])
        load_f32(i + 2, slot).start()
        return carry

    lax.fori_loop(0, nr0, phase0_res, 0)

    def phase0_str(i, carry):
        slot = lax.rem(i, 2)
        load_f32(i, slot).wait()

        @pl.when(i >= nr0 + 2)
        def _():
            store_b(i - 2, slot).wait()

        sb_buf[slot] = lf_buf[slot].astype(jnp.bfloat16)
        store_b(i, slot).start()
        p0_step(i, sb_buf[slot])

        @pl.when(i + 2 < nm0)
        def _():
            load_f32(i + 2, slot).start()

        @pl.when(i >= nm0 - _NSLOT)
        def _():
            load_b(i - (nm0 - _NSLOT), lax.rem(i - (nm0 - _NSLOT), _NSLOT)).start()

        return carry

    lax.fori_loop(nr0, nm0, phase0_str, 0)
    store_b(nm0 - 2, 0).wait()
    store_b(nm0 - 1, 1).wait()

    def p1_step(j, lband):
        z = jnp.dot(lband, x1b_ref[...], preferred_element_type=jnp.float32)
        x0band = x0b_ref[pl.ds(j * bm, bm), :].astype(jnp.float32)
        x2b_ref[pl.ds(j * bm, bm), :] = (2.0 * z - x0band).astype(jnp.bfloat16)

    rat = ns // nr

    def phase1_str(js, carry):
        slot = lax.rem(js, _NSLOT)
        load_b(js, slot).wait()
        p1_step(nr + js, lb_buf[slot])
        load_b(lax.rem(js + _NSLOT, ns), slot).start()

        jr = lax.div(js, rat)

        @pl.when((lax.rem(js, rat) == 0) & (jr < nr))
        def _():
            p1_step(jr, lbr_ref[pl.ds(jr * bm, bm), :])

        return carry

    lax.fori_loop(0, ns, phase1_str, 0)

    def p2_step(j, lband):
        z = jnp.dot(lband, x2b_ref[...], preferred_element_type=jnp.float32)
        x1band = x1b_ref[pl.ds(j * bm, bm), :]
        x3 = 2.0 * z - x1band.astype(jnp.float32)
        t = jnp.dot(x0b_ref[pl.ds(j * bm, bm), :], w_ref[0],
                    preferred_element_type=jnp.float32)
        t = t + jnp.dot(x1band, w_ref[1], preferred_element_type=jnp.float32)
        t = t + jnp.dot(x2b_ref[pl.ds(j * bm, bm), :], w_ref[2],
                        preferred_element_type=jnp.float32)
        t = t + jnp.dot(x3.astype(jnp.bfloat16), w_ref[3],
                        preferred_element_type=jnp.float32)
        t = jnp.maximum(t + b_ref[...], 0.0)
        t = jnp.max(t.reshape(bm // pool, pool, f_out), axis=1)
        out_ref[pl.ds(j * (bm // pool), bm // pool), :] = t

    def phase2_str(js, carry):
        slot = lax.rem(js, _NSLOT)
        load_b(js, slot).wait()
        p2_step(nr + js, lb_buf[slot])

        @pl.when(js + _NSLOT < ns)
        def _():
            load_b(js + _NSLOT, slot).start()

        jr = lax.div(js, rat)

        @pl.when((lax.rem(js, rat) == 0) & (jr < nr))
        def _():
            p2_step(jr, lbr_ref[pl.ds(jr * bm, bm), :])

        return carry

    lax.fori_loop(0, ns, phase2_str, 0)


def kernel(x, laplacian, weight, bias):
    B, N, F_in = x.shape
    F_out = weight.shape[-1]
    degree = weight.shape[0] // F_in
    pool = 2

    x0 = jnp.transpose(x, (1, 2, 0)).reshape(N, F_in * B)
    c = x0.shape[1]
    x0b = x0.astype(jnp.bfloat16)
    w4 = jnp.transpose(weight.reshape(F_in, degree, F_out), (1, 0, 2))
    w4 = w4.astype(jnp.bfloat16)
    b2 = bias.reshape(1, F_out)

    out, _ = pl.pallas_call(
        functools.partial(_fused_kernel, n=N, kr=_KR, bm0=_BM0, bm=_BM,
                          f_out=F_out, pool=pool),
        compiler_params=pltpu.CompilerParams(
            vmem_limit_bytes=110 * 1024 * 1024),
        in_specs=[
            pl.BlockSpec(memory_space=pltpu.MemorySpace.HBM),
            pl.BlockSpec(memory_space=pltpu.MemorySpace.VMEM),
            pl.BlockSpec(memory_space=pltpu.MemorySpace.VMEM),
            pl.BlockSpec(memory_space=pltpu.MemorySpace.VMEM),
        ],
        out_specs=[
            pl.BlockSpec(memory_space=pltpu.MemorySpace.VMEM),
            pl.BlockSpec(memory_space=pltpu.MemorySpace.HBM),
        ],
        out_shape=[
            jax.ShapeDtypeStruct((N // pool, F_out), jnp.float32),
            jax.ShapeDtypeStruct((N - _KR, N), jnp.bfloat16),
        ],
        scratch_shapes=[
            pltpu.VMEM((N, c), jnp.bfloat16),
            pltpu.VMEM((N, c), jnp.bfloat16),
            pltpu.VMEM((_KR, N), jnp.bfloat16),
            pltpu.VMEM((2, _BM0, N), jnp.float32),
            pltpu.VMEM((2, _BM0, N), jnp.bfloat16),
            pltpu.VMEM((_NSLOT, _BM, N), jnp.bfloat16),
            pltpu.SemaphoreType.DMA((2,)),
            pltpu.SemaphoreType.DMA((2,)),
            pltpu.SemaphoreType.DMA((_NSLOT,)),
        ],
    )(laplacian, x0b, w4, b2)

    return out.reshape(B, N // pool, F_out)


# speedup vs baseline: 1.0973x; 1.0973x over previous
import functools

import jax
import jax.numpy as jnp
from jax import lax
from jax.experimental import pallas as pl
from jax.experimental.pallas import tpu as pltpu

_BM0 = 256
_BM = 256
_NSLOT = 2
_KR = 1024


def _fused_kernel(l_hbm, x0b_ref, w_ref, b_ref, out_ref, lb_hbm,
                  x1b_ref, x2b_ref, lbr_ref, lf_buf, sb_buf, lb_buf,
                  lf_sem, st_sem, lb_sem, *, n, kr, bm0, bm, f_out, pool):
    nm0 = n // bm0
    nr0 = kr // bm0
    nm = n // bm
    nr = kr // bm
    ns = nm - nr

    def load_f32(i, slot):
        return pltpu.make_async_copy(
            l_hbm.at[pl.ds(i * bm0, bm0), :], lf_buf.at[slot],
            lf_sem.at[slot])

    def store_b(i, slot):
        return pltpu.make_async_copy(
            sb_buf.at[slot], lb_hbm.at[pl.ds(i * bm0 - kr, bm0), :],
            st_sem.at[slot])

    def load_b(j, slot):
        return pltpu.make_async_copy(
            lb_hbm.at[pl.ds(j * bm, bm), :], lb_buf.at[slot],
            lb_sem.at[slot])

    load_f32(0, 0).start()
    load_f32(1, 1).start()

    def p0_step(i, lband):
        y = jnp.dot(lband, x0b_ref[...], preferred_element_type=jnp.float32)
        x1b_ref[pl.ds(i * bm0, bm0), :] = y.astype(jnp.bfloat16)

    def phase0_res(i, carry):
        slot = lax.rem(i, 2)
        load_f32(i, slot).wait()
        lbr_ref[pl.ds(i * bm0, bm0), :] = lf_buf[slot].astype(jnp.bfloat16)
        p0_step(i, lbr_ref[pl.ds(i * bm0, bm0), :])
        load_f32(i + 2, slot).start()
        return carry

    lax.fori_loop(0, nr0, phase0_res, 0)

    def phase0_str(i, carry):
        slot = lax.rem(i, 2)
        load_f32(i, slot).wait()

        @pl.when(i >= nr0 + 2)
        def _():
            store_b(i - 2, slot).wait()

        sb_buf[slot] = lf_buf[slot].astype(jnp.bfloat16)
        store_b(i, slot).start()
        p0_step(i, sb_buf[slot])

        @pl.when(i + 2 < nm0)
        def _():
            load_f32(i + 2, slot).start()

        @pl.when(i >= nm0 - _NSLOT)
        def _():
            load_b(i - (nm0 - _NSLOT), lax.rem(i - (nm0 - _NSLOT), _NSLOT)).start()

        return carry

    lax.fori_loop(nr0, nm0, phase0_str, 0)
    store_b(nm0 - 2, 0).wait()
    store_b(nm0 - 1, 1).wait()

    def p1_step(j, lband):
        z = jnp.dot(lband, x1b_ref[...], preferred_element_type=jnp.float32)
        x0band = x0b_ref[pl.ds(j * bm, bm), :].astype(jnp.float32)
        x2b_ref[pl.ds(j * bm, bm), :] = (2.0 * z - x0band).astype(jnp.bfloat16)

    rat = ns // nr

    def phase1_str(js, carry):
        slot = lax.rem(js, _NSLOT)
        load_b(js, slot).wait()
        p1_step(nr + js, lb_buf[slot])
        load_b(lax.rem(js + _NSLOT, ns), slot).start()

        jr = lax.div(js, rat)

        @pl.when((lax.rem(js, rat) == 0) & (jr < nr))
        def _():
            p1_step(jr, lbr_ref[pl.ds(jr * bm, bm), :])

        return carry

    lax.fori_loop(0, ns, phase1_str, 0)

    def p2_step(j, lband):
        z = jnp.dot(lband, x2b_ref[...], preferred_element_type=jnp.float32)
        x1band = x1b_ref[pl.ds(j * bm, bm), :]
        x3 = 2.0 * z - x1band.astype(jnp.float32)
        t = jnp.dot(x0b_ref[pl.ds(j * bm, bm), :], w_ref[0],
                    preferred_element_type=jnp.float32)
        t = t + jnp.dot(x1band, w_ref[1], preferred_element_type=jnp.float32)
        t = t + jnp.dot(x2b_ref[pl.ds(j * bm, bm), :], w_ref[2],
                        preferred_element_type=jnp.float32)
        t = t + jnp.dot(x3.astype(jnp.bfloat16), w_ref[3],
                        preferred_element_type=jnp.float32)
        t = jnp.maximum(t + b_ref[...], 0.0)
        t = jnp.max(t.reshape(bm // pool, pool, f_out), axis=1)
        out_ref[pl.ds(j * (bm // pool), bm // pool), :] = t

    def phase2_str(js, carry):
        slot = lax.rem(js, _NSLOT)
        load_b(js, slot).wait()
        p2_step(nr + js, lb_buf[slot])

        @pl.when(js + _NSLOT < ns)
        def _():
            load_b(js + _NSLOT, slot).start()

        jr = lax.div(js, rat)

        @pl.when((lax.rem(js, rat) == 0) & (jr < nr))
        def _():
            p2_step(jr, lbr_ref[pl.ds(jr * bm, bm), :])

        return carry

    lax.fori_loop(0, ns, phase2_str, 0)


def kernel(x, laplacian, weight, bias):
    B, N, F_in = x.shape
    F_out = weight.shape[-1]
    degree = weight.shape[0] // F_in
    pool = 2

    x0 = jnp.transpose(x, (1, 2, 0)).reshape(N, F_in * B)
    c = x0.shape[1]
    x0b = x0.astype(jnp.bfloat16)
    w4 = jnp.transpose(weight.reshape(F_in, degree, F_out), (1, 0, 2))
    w4 = w4.astype(jnp.bfloat16)
    b2 = bias.reshape(1, F_out)

    out, _ = pl.pallas_call(
        functools.partial(_fused_kernel, n=N, kr=_KR, bm0=_BM0, bm=_BM,
                          f_out=F_out, pool=pool),
        compiler_params=pltpu.CompilerParams(
            vmem_limit_bytes=110 * 1024 * 1024),
        in_specs=[
            pl.BlockSpec(memory_space=pltpu.MemorySpace.HBM),
            pl.BlockSpec(memory_space=pltpu.MemorySpace.VMEM),
            pl.BlockSpec(memory_space=pltpu.MemorySpace.VMEM),
            pl.BlockSpec(memory_space=pltpu.MemorySpace.VMEM),
        ],
        out_specs=[
            pl.BlockSpec(memory_space=pltpu.MemorySpace.VMEM),
            pl.BlockSpec(memory_space=pltpu.MemorySpace.HBM),
        ],
        out_shape=[
            jax.ShapeDtypeStruct((N // pool, F_out), jnp.float32),
            jax.ShapeDtypeStruct((N - _KR, N), jnp.bfloat16),
        ],
        scratch_shapes=[
            pltpu.VMEM((N, c), jnp.bfloat16),
            pltpu.VMEM((N, c), jnp.bfloat16),
            pltpu.VMEM((_KR, N), jnp.bfloat16),
            pltpu.VMEM((2, _BM0, N), jnp.float32),
            pltpu.VMEM((2, _BM0, N), jnp.bfloat16),
            pltpu.VMEM((_NSLOT, _BM, N), jnp.bfloat16),
            pltpu.SemaphoreType.DMA((2,)),
            pltpu.SemaphoreType.DMA((2,)),
            pltpu.SemaphoreType.DMA((_NSLOT,)),
        ],
    )(laplacian, x0b, w4, b2)

    return out.reshape(B, N // pool, F_out)


# speedup vs baseline: 1.1186x; 1.0195x over previous
import functools

import jax
import jax.numpy as jnp
from jax import lax
from jax.experimental import pallas as pl
from jax.experimental.pallas import tpu as pltpu

_BM0 = 128
_BM = 256
_NSLOT = 2
_KR = 2048


def _fused_kernel(l_hbm, x0b_ref, w_ref, b_ref, out_ref, lb_hbm,
                  x1b_ref, x2b_ref, lbr_ref, lf_buf, sb_buf, lb_buf,
                  lf_sem, st_sem, lb_sem, *, n, kr, bm0, bm, f_out, pool):
    nm0 = n // bm0
    nr0 = kr // bm0
    nm = n // bm
    nr = kr // bm
    ns = nm - nr

    def load_f32(i, slot):
        return pltpu.make_async_copy(
            l_hbm.at[pl.ds(i * bm0, bm0), :], lf_buf.at[slot],
            lf_sem.at[slot])

    def store_b(i, slot):
        return pltpu.make_async_copy(
            sb_buf.at[slot], lb_hbm.at[pl.ds(i * bm0 - kr, bm0), :],
            st_sem.at[slot])

    def load_b(j, slot):
        return pltpu.make_async_copy(
            lb_hbm.at[pl.ds(j * bm, bm), :], lb_buf.at[slot],
            lb_sem.at[slot])

    load_f32(0, 0).start()
    load_f32(1, 1).start()

    def p0_step(i, lband):
        y = jnp.dot(lband, x0b_ref[...], preferred_element_type=jnp.float32)
        x1b_ref[pl.ds(i * bm0, bm0), :] = y.astype(jnp.bfloat16)

    def phase0_res(i, carry):
        slot = lax.rem(i, 2)
        load_f32(i, slot).wait()
        lbr_ref[pl.ds(i * bm0, bm0), :] = lf_buf[slot].astype(jnp.bfloat16)
        p0_step(i, lbr_ref[pl.ds(i * bm0, bm0), :])
        load_f32(i + 2, slot).start()
        return carry

    lax.fori_loop(0, nr0, phase0_res, 0)

    def phase0_str(i, carry):
        slot = lax.rem(i, 2)
        load_f32(i, slot).wait()

        @pl.when(i >= nr0 + 2)
        def _():
            store_b(i - 2, slot).wait()

        sb_buf[slot] = lf_buf[slot].astype(jnp.bfloat16)
        store_b(i, slot).start()
        p0_step(i, sb_buf[slot])

        @pl.when(i + 2 < nm0)
        def _():
            load_f32(i + 2, slot).start()

        @pl.when(i >= nm0 - _NSLOT)
        def _():
            load_b(i - (nm0 - _NSLOT), lax.rem(i - (nm0 - _NSLOT), _NSLOT)).start()

        return carry

    lax.fori_loop(nr0, nm0, phase0_str, 0)
    store_b(nm0 - 2, 0).wait()
    store_b(nm0 - 1, 1).wait()

    def p1_step(j, lband):
        z = jnp.dot(lband, x1b_ref[...], preferred_element_type=jnp.float32)
        x0band = x0b_ref[pl.ds(j * bm, bm), :].astype(jnp.float32)
        x2b_ref[pl.ds(j * bm, bm), :] = (2.0 * z - x0band).astype(jnp.bfloat16)

    rat = ns // nr

    def phase1_str(js, carry):
        slot = lax.rem(js, _NSLOT)
        jr = lax.div(js, rat)

        @pl.when((lax.rem(js, rat) == 0) & (jr < nr))
        def _():
            p1_step(jr, lbr_ref[pl.ds(jr * bm, bm), :])

        load_b(js, slot).wait()
        p1_step(nr + js, lb_buf[slot])
        load_b(lax.rem(js + _NSLOT, ns), slot).start()
        return carry

    lax.fori_loop(0, ns, phase1_str, 0)

    def p2_step(j, lband):
        z = jnp.dot(lband, x2b_ref[...], preferred_element_type=jnp.float32)
        x1band = x1b_ref[pl.ds(j * bm, bm), :]
        x3 = 2.0 * z - x1band.astype(jnp.float32)
        t = jnp.dot(x0b_ref[pl.ds(j * bm, bm), :], w_ref[0],
                    preferred_element_type=jnp.float32)
        t = t + jnp.dot(x1band, w_ref[1], preferred_element_type=jnp.float32)
        t = t + jnp.dot(x2b_ref[pl.ds(j * bm, bm), :], w_ref[2],
                        preferred_element_type=jnp.float32)
        t = t + jnp.dot(x3.astype(jnp.bfloat16), w_ref[3],
                        preferred_element_type=jnp.float32)
        t = jnp.maximum(t + b_ref[...], 0.0)
        t = jnp.max(t.reshape(bm // pool, pool, f_out), axis=1)
        out_ref[pl.ds(j * (bm // pool), bm // pool), :] = t

    def phase2_str(js, carry):
        slot = lax.rem(js, _NSLOT)
        jr = lax.div(js, rat)

        @pl.when((lax.rem(js, rat) == 0) & (jr < nr))
        def _():
            p2_step(jr, lbr_ref[pl.ds(jr * bm, bm), :])

        load_b(js, slot).wait()
        p2_step(nr + js, lb_buf[slot])

        @pl.when(js + _NSLOT < ns)
        def _():
            load_b(js + _NSLOT, slot).start()

        return carry

    lax.fori_loop(0, ns, phase2_str, 0)


def kernel(x, laplacian, weight, bias):
    B, N, F_in = x.shape
    F_out = weight.shape[-1]
    degree = weight.shape[0] // F_in
    pool = 2

    x0 = jnp.transpose(x, (1, 2, 0)).reshape(N, F_in * B)
    c = x0.shape[1]
    x0b = x0.astype(jnp.bfloat16)
    w4 = jnp.transpose(weight.reshape(F_in, degree, F_out), (1, 0, 2))
    w4 = w4.astype(jnp.bfloat16)
    b2 = bias.reshape(1, F_out)

    out, _ = pl.pallas_call(
        functools.partial(_fused_kernel, n=N, kr=_KR, bm0=_BM0, bm=_BM,
                          f_out=F_out, pool=pool),
        compiler_params=pltpu.CompilerParams(
            vmem_limit_bytes=110 * 1024 * 1024),
        in_specs=[
            pl.BlockSpec(memory_space=pltpu.MemorySpace.HBM),
            pl.BlockSpec(memory_space=pltpu.MemorySpace.VMEM),
            pl.BlockSpec(memory_space=pltpu.MemorySpace.VMEM),
            pl.BlockSpec(memory_space=pltpu.MemorySpace.VMEM),
        ],
        out_specs=[
            pl.BlockSpec(memory_space=pltpu.MemorySpace.VMEM),
            pl.BlockSpec(memory_space=pltpu.MemorySpace.HBM),
        ],
        out_shape=[
            jax.ShapeDtypeStruct((N // pool, F_out), jnp.float32),
            jax.ShapeDtypeStruct((N - _KR, N), jnp.bfloat16),
        ],
        scratch_shapes=[
            pltpu.VMEM((N, c), jnp.bfloat16),
            pltpu.VMEM((N, c), jnp.bfloat16),
            pltpu.VMEM((_KR, N), jnp.bfloat16),
            pltpu.VMEM((2, _BM0, N), jnp.float32),
            pltpu.VMEM((2, _BM0, N), jnp.bfloat16),
            pltpu.VMEM((_NSLOT, _BM, N), jnp.bfloat16),
            pltpu.SemaphoreType.DMA((2,)),
            pltpu.SemaphoreType.DMA((2,)),
            pltpu.SemaphoreType.DMA((_NSLOT,)),
        ],
    )(laplacian, x0b, w4, b2)

    return out.reshape(B, N // pool, F_out)


# speedup vs baseline: 1.1571x; 1.0344x over previous
import functools

import jax
import jax.numpy as jnp
from jax import lax
from jax.experimental import pallas as pl
from jax.experimental.pallas import tpu as pltpu

_BM0 = 128
_BM = 256
_NSLOT = 2
_KR = 2048


def _fused_kernel(l_hbm, x0b_ref, w_ref, b_ref, out_ref, lb_hbm,
                  x1b_ref, x2b_ref, lbr_ref, lf_buf, sb_buf, lb_buf,
                  z1r_ref, lf_sem, st_sem, lb_sem,
                  *, n, kr, bm0, bm, f_out, pool):
    nm0 = n // bm0
    nr0 = kr // bm0
    nm = n // bm
    nr = kr // bm
    ns = nm - nr

    def load_f32(i, slot):
        return pltpu.make_async_copy(
            l_hbm.at[pl.ds(i * bm0, bm0), :], lf_buf.at[slot],
            lf_sem.at[slot])

    def store_b(i, slot):
        return pltpu.make_async_copy(
            sb_buf.at[slot], lb_hbm.at[pl.ds(i * bm0 - kr, bm0), :],
            st_sem.at[slot])

    def load_b(j, slot):
        return pltpu.make_async_copy(
            lb_hbm.at[pl.ds(j * bm, bm), :], lb_buf.at[slot],
            lb_sem.at[slot])

    load_f32(0, 0).start()
    load_f32(1, 1).start()

    def p0_step(i, lband):
        y = jnp.dot(lband, x0b_ref[...], preferred_element_type=jnp.float32)
        x1b_ref[pl.ds(i * bm0, bm0), :] = y.astype(jnp.bfloat16)

    def phase0_res(i, carry):
        slot = lax.rem(i, 2)
        load_f32(i, slot).wait()
        lbr_ref[pl.ds(i * bm0, bm0), :] = lf_buf[slot].astype(jnp.bfloat16)
        p0_step(i, lbr_ref[pl.ds(i * bm0, bm0), :])
        load_f32(i + 2, slot).start()
        return carry

    lax.fori_loop(0, nr0, phase0_res, 0)
    z1r_ref[...] = jnp.zeros(z1r_ref.shape, z1r_ref.dtype)

    def z1r_acc(kb):
        z1r_ref[...] = z1r_ref[...] + jnp.dot(
            lbr_ref[:, pl.ds(kb * bm0, bm0)],
            x1b_ref[pl.ds(kb * bm0, bm0), :],
            preferred_element_type=jnp.float32)

    def phase0_str(i, carry):
        slot = lax.rem(i, 2)
        load_f32(i, slot).wait()

        @pl.when(i >= nr0 + 2)
        def _():
            store_b(i - 2, slot).wait()

        sb_buf[slot] = lf_buf[slot].astype(jnp.bfloat16)
        store_b(i, slot).start()
        p0_step(i, sb_buf[slot])

        @pl.when(i + 2 < nm0)
        def _():
            load_f32(i + 2, slot).start()

        @pl.when(i >= nm0 - _NSLOT)
        def _():
            load_b(i - (nm0 - _NSLOT), lax.rem(i - (nm0 - _NSLOT), _NSLOT)).start()

        z1r_acc(i)

        @pl.when(i < 2 * nr0)
        def _():
            z1r_acc(i - nr0)

        return carry

    lax.fori_loop(nr0, nm0, phase0_str, 0)
    store_b(nm0 - 2, 0).wait()
    store_b(nm0 - 1, 1).wait()
    x2b_ref[:kr, :] = (2.0 * z1r_ref[...]
                       - x0b_ref[:kr, :].astype(jnp.float32)
                       ).astype(jnp.bfloat16)

    def p1_step(j, lband):
        z = jnp.dot(lband, x1b_ref[...], preferred_element_type=jnp.float32)
        x0band = x0b_ref[pl.ds(j * bm, bm), :].astype(jnp.float32)
        x2b_ref[pl.ds(j * bm, bm), :] = (2.0 * z - x0band).astype(jnp.bfloat16)

    rat = ns // nr

    def phase1_str(js, carry):
        slot = lax.rem(js, _NSLOT)
        load_b(js, slot).wait()
        p1_step(nr + js, lb_buf[slot])
        load_b(lax.rem(js + _NSLOT, ns), slot).start()
        return carry

    lax.fori_loop(0, ns, phase1_str, 0)

    def p2_step(j, lband):
        z = jnp.dot(lband, x2b_ref[...], preferred_element_type=jnp.float32)
        x1band = x1b_ref[pl.ds(j * bm, bm), :]
        x3 = 2.0 * z - x1band.astype(jnp.float32)
        t = jnp.dot(x0b_ref[pl.ds(j * bm, bm), :], w_ref[0],
                    preferred_element_type=jnp.float32)
        t = t + jnp.dot(x1band, w_ref[1], preferred_element_type=jnp.float32)
        t = t + jnp.dot(x2b_ref[pl.ds(j * bm, bm), :], w_ref[2],
                        preferred_element_type=jnp.float32)
        t = t + jnp.dot(x3.astype(jnp.bfloat16), w_ref[3],
                        preferred_element_type=jnp.float32)
        t = jnp.maximum(t + b_ref[...], 0.0)
        t = jnp.max(t.reshape(bm // pool, pool, f_out), axis=1)
        out_ref[pl.ds(j * (bm // pool), bm // pool), :] = t

    def phase2_str(js, carry):
        slot = lax.rem(js, _NSLOT)
        jr = lax.div(js, rat)

        @pl.when((lax.rem(js, rat) == 0) & (jr < nr))
        def _():
            p2_step(jr, lbr_ref[pl.ds(jr * bm, bm), :])

        load_b(js, slot).wait()
        p2_step(nr + js, lb_buf[slot])

        @pl.when(js + _NSLOT < ns)
        def _():
            load_b(js + _NSLOT, slot).start()

        return carry

    lax.fori_loop(0, ns, phase2_str, 0)


def kernel(x, laplacian, weight, bias):
    B, N, F_in = x.shape
    F_out = weight.shape[-1]
    degree = weight.shape[0] // F_in
    pool = 2

    x0 = jnp.transpose(x, (1, 2, 0)).reshape(N, F_in * B)
    c = x0.shape[1]
    x0b = x0.astype(jnp.bfloat16)
    w4 = jnp.transpose(weight.reshape(F_in, degree, F_out), (1, 0, 2))
    w4 = w4.astype(jnp.bfloat16)
    b2 = bias.reshape(1, F_out)

    out, _ = pl.pallas_call(
        functools.partial(_fused_kernel, n=N, kr=_KR, bm0=_BM0, bm=_BM,
                          f_out=F_out, pool=pool),
        compiler_params=pltpu.CompilerParams(
            vmem_limit_bytes=110 * 1024 * 1024),
        in_specs=[
            pl.BlockSpec(memory_space=pltpu.MemorySpace.HBM),
            pl.BlockSpec(memory_space=pltpu.MemorySpace.VMEM),
            pl.BlockSpec(memory_space=pltpu.MemorySpace.VMEM),
            pl.BlockSpec(memory_space=pltpu.MemorySpace.VMEM),
        ],
        out_specs=[
            pl.BlockSpec(memory_space=pltpu.MemorySpace.VMEM),
            pl.BlockSpec(memory_space=pltpu.MemorySpace.HBM),
        ],
        out_shape=[
            jax.ShapeDtypeStruct((N // pool, F_out), jnp.float32),
            jax.ShapeDtypeStruct((N - _KR, N), jnp.bfloat16),
        ],
        scratch_shapes=[
            pltpu.VMEM((N, c), jnp.bfloat16),
            pltpu.VMEM((N, c), jnp.bfloat16),
            pltpu.VMEM((_KR, N), jnp.bfloat16),
            pltpu.VMEM((2, _BM0, N), jnp.float32),
            pltpu.VMEM((2, _BM0, N), jnp.bfloat16),
            pltpu.VMEM((_NSLOT, _BM, N), jnp.bfloat16),
            pltpu.VMEM((_KR, 64), jnp.float32),
            pltpu.SemaphoreType.DMA((2,)),
            pltpu.SemaphoreType.DMA((2,)),
            pltpu.SemaphoreType.DMA((_NSLOT,)),
        ],
    )(laplacian, x0b, w4, b2)

    return out.reshape(B, N // pool, F_out)


# speedup vs baseline: 1.1578x; 1.0005x over previous
import functools

import jax
import jax.numpy as jnp
from jax import lax
from jax.experimental import pallas as pl
from jax.experimental.pallas import tpu as pltpu

_BM0 = 128
_BM = 256
_NSLOT = 2
_KR = 2048


def _fused_kernel(l_hbm, x0b_ref, w_ref, b_ref, out_ref, lb_hbm,
                  x1b_ref, x2b_ref, lbr_ref, lf_buf, sb_buf, lb_buf,
                  z1r_ref, lf_sem, st_sem, lb_sem,
                  *, n, kr, bm0, bm, f_out, pool):
    nm0 = n // bm0
    nr0 = kr // bm0
    nm = n // bm
    nr = kr // bm
    ns = nm - nr

    def load_f32(i, slot):
        return pltpu.make_async_copy(
            l_hbm.at[pl.ds(i * bm0, bm0), :], lf_buf.at[slot],
            lf_sem.at[slot])

    def store_b(i, slot):
        return pltpu.make_async_copy(
            sb_buf.at[slot], lb_hbm.at[pl.ds(i * bm0 - kr, bm0), :],
            st_sem.at[slot])

    def load_b(j, slot):
        return pltpu.make_async_copy(
            lb_hbm.at[pl.ds(j * bm, bm), :], lb_buf.at[slot],
            lb_sem.at[slot])

    load_f32(0, 0).start()
    load_f32(1, 1).start()

    def p0_step(i, lband):
        y = jnp.dot(lband, x0b_ref[...], preferred_element_type=jnp.float32)
        x1b_ref[pl.ds(i * bm0, bm0), :] = y.astype(jnp.bfloat16)

    def phase0_res(i, carry):
        slot = lax.rem(i, 2)
        load_f32(i, slot).wait()
        lbr_ref[pl.ds(i * bm0, bm0), :] = lf_buf[slot].astype(jnp.bfloat16)
        p0_step(i, lbr_ref[pl.ds(i * bm0, bm0), :])
        load_f32(i + 2, slot).start()
        return carry

    lax.fori_loop(0, nr0, phase0_res, 0)
    z1r_ref[...] = jnp.zeros(z1r_ref.shape, z1r_ref.dtype)

    def z1r_acc(kb):
        z1r_ref[...] = z1r_ref[...] + jnp.dot(
            lbr_ref[:, pl.ds(kb * bm0, bm0)],
            x1b_ref[pl.ds(kb * bm0, bm0), :],
            preferred_element_type=jnp.float32)

    def phase0_str(i, carry):
        slot = lax.rem(i, 2)
        load_f32(i, slot).wait()

        @pl.when(i >= nr0 + 2)
        def _():
            store_b(i - 2, slot).wait()

        sb_buf[slot] = lf_buf[slot].astype(jnp.bfloat16)
        store_b(i, slot).start()
        p0_step(i, sb_buf[slot])

        @pl.when(i + 2 < nm0)
        def _():
            load_f32(i + 2, slot).start()

        @pl.when(i >= nm0 - _NSLOT)
        def _():
            load_b(i - (nm0 - _NSLOT), lax.rem(i - (nm0 - _NSLOT), _NSLOT)).start()

        z1r_acc(i)

        @pl.when(i < 2 * nr0)
        def _():
            z1r_acc(i - nr0)

        return carry

    lax.fori_loop(nr0, nm0, phase0_str, 0)
    store_b(nm0 - 2, 0).wait()
    store_b(nm0 - 1, 1).wait()
    x2b_ref[:kr, :] = (2.0 * z1r_ref[...]
                       - x0b_ref[:kr, :].astype(jnp.float32)
                       ).astype(jnp.bfloat16)

    def p1_step(j, lband):
        z = jnp.dot(lband, x1b_ref[...], preferred_element_type=jnp.float32)
        x0band = x0b_ref[pl.ds(j * bm, bm), :].astype(jnp.float32)
        x2b_ref[pl.ds(j * bm, bm), :] = (2.0 * z - x0band).astype(jnp.bfloat16)

    rat = ns // nr

    def phase1_str(js, carry):
        slot = lax.rem(js, _NSLOT)
        load_b(js, slot).wait()
        p1_step(nr + js, lb_buf[slot])
        load_b(lax.rem(js + _NSLOT, ns), slot).start()
        return carry

    lax.fori_loop(0, ns, phase1_str, 0)

    def p2_step(j, lband):
        z = jnp.dot(lband, x2b_ref[...], preferred_element_type=jnp.float32)
        t = jnp.dot(x0b_ref[pl.ds(j * bm, bm), :], w_ref[0],
                    preferred_element_type=jnp.float32)
        t = t + jnp.dot(x1b_ref[pl.ds(j * bm, bm), :], w_ref[1],
                        preferred_element_type=jnp.float32)
        t = t + jnp.dot(x2b_ref[pl.ds(j * bm, bm), :], w_ref[2],
                        preferred_element_type=jnp.float32)
        t = t + jnp.dot(z.astype(jnp.bfloat16), w_ref[3],
                        preferred_element_type=jnp.float32)
        t = jnp.maximum(t + b_ref[...], 0.0)
        t = jnp.max(t.reshape(bm // pool, pool, f_out), axis=1)
        out_ref[pl.ds(j * (bm // pool), bm // pool), :] = t

    def phase2_str(js, carry):
        slot = lax.rem(js, _NSLOT)
        jr = lax.div(js, rat)

        @pl.when((lax.rem(js, rat) == 0) & (jr < nr))
        def _():
            p2_step(jr, lbr_ref[pl.ds(jr * bm, bm), :])

        load_b(js, slot).wait()
        p2_step(nr + js, lb_buf[slot])

        @pl.when(js + _NSLOT < ns)
        def _():
            load_b(js + _NSLOT, slot).start()

        return carry

    lax.fori_loop(0, ns, phase2_str, 0)


def kernel(x, laplacian, weight, bias):
    B, N, F_in = x.shape
    F_out = weight.shape[-1]
    degree = weight.shape[0] // F_in
    pool = 2

    x0 = jnp.transpose(x, (1, 2, 0)).reshape(N, F_in * B)
    c = x0.shape[1]
    x0b = x0.astype(jnp.bfloat16)
    w4 = jnp.transpose(weight.reshape(F_in, degree, F_out), (1, 0, 2))
    w4 = jnp.stack([w4[0], w4[1] - w4[3], w4[2], 2.0 * w4[3]])
    w4 = w4.astype(jnp.bfloat16)
    b2 = bias.reshape(1, F_out)

    out, _ = pl.pallas_call(
        functools.partial(_fused_kernel, n=N, kr=_KR, bm0=_BM0, bm=_BM,
                          f_out=F_out, pool=pool),
        compiler_params=pltpu.CompilerParams(
            vmem_limit_bytes=110 * 1024 * 1024),
        in_specs=[
            pl.BlockSpec(memory_space=pltpu.MemorySpace.HBM),
            pl.BlockSpec(memory_space=pltpu.MemorySpace.VMEM),
            pl.BlockSpec(memory_space=pltpu.MemorySpace.VMEM),
            pl.BlockSpec(memory_space=pltpu.MemorySpace.VMEM),
        ],
        out_specs=[
            pl.BlockSpec(memory_space=pltpu.MemorySpace.VMEM),
            pl.BlockSpec(memory_space=pltpu.MemorySpace.HBM),
        ],
        out_shape=[
            jax.ShapeDtypeStruct((N // pool, F_out), jnp.float32),
            jax.ShapeDtypeStruct((N - _KR, N), jnp.bfloat16),
        ],
        scratch_shapes=[
            pltpu.VMEM((N, c), jnp.bfloat16),
            pltpu.VMEM((N, c), jnp.bfloat16),
            pltpu.VMEM((_KR, N), jnp.bfloat16),
            pltpu.VMEM((2, _BM0, N), jnp.float32),
            pltpu.VMEM((2, _BM0, N), jnp.bfloat16),
            pltpu.VMEM((_NSLOT, _BM, N), jnp.bfloat16),
            pltpu.VMEM((_KR, 64), jnp.float32),
            pltpu.SemaphoreType.DMA((2,)),
            pltpu.SemaphoreType.DMA((2,)),
            pltpu.SemaphoreType.DMA((_NSLOT,)),
        ],
    )(laplacian, x0b, w4, b2)

    return out.reshape(B, N // pool, F_out)
